# Initial kernel scaffold; baseline (speedup 1.0000x reference)
import jax
import jax.numpy as jnp
from jax.experimental import pallas as pl


def kernel(x, W1, b1, W2, b2, W3, b3, W4, b4, W5, b5, W6, b6, Wr1, br1, Wr2, br2):
    raise NotImplementedError("write your pallas kernel here")


# speedup vs baseline: 1.8000x; 1.8000x over previous
import jax
import jax.numpy as jnp
from jax.experimental import pallas as pl

B = 4096
D_IN = 784
H = 2048
D_OUT = 1024
RH = 256
NP = 512
BM = 256


def _mlp_body(x_ref, W1, b1, W2, b2, W3, b3, W4, b4, W5, b5, W6, b6,
              Wr1, br1, Wr2, br2, out_ref):
    f32 = jnp.float32
    bf16 = jnp.bfloat16
    x = x_ref[...]

    r = jnp.dot(x.astype(bf16), Wr1[...], preferred_element_type=f32) + br1[...]
    r = jnp.maximum(r, 0.0)
    logits = jnp.dot(r.astype(bf16), Wr2[...], preferred_element_type=f32) + br2[...]
    m = jnp.max(logits, axis=1, keepdims=True)
    e = jnp.exp(logits - m)
    probs = e / jnp.sum(e, axis=1, keepdims=True)

    lane = jax.lax.broadcasted_iota(jnp.int32, (BM, NP), 1)

    def gsum(mask):
        return jnp.sum(jnp.where(mask, probs, 0.0), axis=1, keepdims=True)

    g_in = [gsum(lane // 128 == i) for i in range(4)]
    g1 = [gsum((lane // 64) % 2 == j) for j in range(2)]
    g2 = [gsum((lane // 32) % 2 == j) for j in range(2)]
    g3 = [gsum((lane // 16) % 2 == j) for j in range(2)]
    g4 = [gsum((lane // 8) % 2 == j) for j in range(2)]
    g5 = [gsum((lane // 4) % 2 == j) for j in range(2)]
    g_out = [gsum(lane % 4 == o) for o in range(4)]

    pix = jax.lax.broadcasted_iota(jnp.int32, (BM, D_IN), 1)
    quad = (pix // 28 >= 14).astype(jnp.int32) * 2 + (pix % 28 >= 14).astype(jnp.int32)
    gin_full = (jnp.where(quad == 0, g_in[0], 0.0) + jnp.where(quad == 1, g_in[1], 0.0)
                + jnp.where(quad == 2, g_in[2], 0.0) + jnp.where(quad == 3, g_in[3], 0.0))
    xg = x * gin_full

    def gate2(ga, gb, ncols):
        idx = jax.lax.broadcasted_iota(jnp.int32, (BM, ncols), 1)
        return jnp.where(idx < ncols // 2, ga, gb)

    def layer(h, W, b, gfull, act=True):
        y = jnp.dot(h.astype(bf16), W[...], preferred_element_type=f32) + b[...]
        if act:
            y = jnp.maximum(y, 0.0)
        return y * gfull

    h = layer(xg, W1, b1, gate2(g1[0], g1[1], H))
    h = layer(h, W2, b2, gate2(g2[0], g2[1], H))
    h = layer(h, W3, b3, gate2(g3[0], g3[1], H))
    h = layer(h, W4, b4, gate2(g4[0], g4[1], H))
    h = layer(h, W5, b5, gate2(g5[0], g5[1], H))

    oidx = jax.lax.broadcasted_iota(jnp.int32, (BM, D_OUT), 1)
    gout_full = (jnp.where(oidx < 256, g_out[0], 0.0)
                 + jnp.where((oidx >= 256) & (oidx < 512), g_out[1], 0.0)
                 + jnp.where((oidx >= 512) & (oidx < 768), g_out[2], 0.0)
                 + jnp.where(oidx >= 768, g_out[3], 0.0))
    out_ref[...] = layer(h, W6, b6, gout_full, act=False)


def kernel(x, W1, b1, W2, b2, W3, b3, W4, b4, W5, b5, W6, b6, Wr1, br1, Wr2, br2):
    wb = lambda w: w.astype(jnp.bfloat16)
    bb = lambda b: b.reshape(1, -1)

    def full(arr):
        return pl.BlockSpec(arr.shape, lambda i: (0, 0))

    ops = [wb(W1), bb(b1), wb(W2), bb(b2), wb(W3), bb(b3), wb(W4), bb(b4),
           wb(W5), bb(b5), wb(W6), bb(b6), wb(Wr1), bb(br1), wb(Wr2), bb(br2)]

    return pl.pallas_call(
        _mlp_body,
        grid=(B // BM,),
        in_specs=[pl.BlockSpec((BM, D_IN), lambda i: (i, 0))] + [full(a) for a in ops],
        out_specs=pl.BlockSpec((BM, D_OUT), lambda i: (i, 0)),
        out_shape=jax.ShapeDtypeStruct((B, D_OUT), jnp.float32),
    )(x, *ops)


# speedup vs baseline: 1.8570x; 1.0317x over previous
import jax
import jax.numpy as jnp
from jax.experimental import pallas as pl

B = 4096
D_IN = 784
H = 2048
D_OUT = 1024
RH = 256
NP = 512
BM = 512


def _mlp_body(x_ref, W1, b1, W2, b2, W3, b3, W4, b4, W5, b5, W6, b6,
              Wr1, br1, Wr2, br2, out_ref):
    f32 = jnp.float32
    bf16 = jnp.bfloat16
    x = x_ref[...]

    r = jnp.dot(x.astype(bf16), Wr1[...], preferred_element_type=f32) + br1[...]
    r = jnp.maximum(r, 0.0)
    logits = jnp.dot(r.astype(bf16), Wr2[...], preferred_element_type=f32) + br2[...]
    m = jnp.max(logits, axis=1, keepdims=True)
    e = jnp.exp(logits - m)
    probs = e / jnp.sum(e, axis=1, keepdims=True)

    lane = jax.lax.broadcasted_iota(jnp.int32, (BM, NP), 1)

    def gsum(mask):
        return jnp.sum(jnp.where(mask, probs, 0.0), axis=1, keepdims=True)

    g_in = [gsum(lane // 128 == i) for i in range(4)]
    g1 = [gsum((lane // 64) % 2 == j) for j in range(2)]
    g2 = [gsum((lane // 32) % 2 == j) for j in range(2)]
    g3 = [gsum((lane // 16) % 2 == j) for j in range(2)]
    g4 = [gsum((lane // 8) % 2 == j) for j in range(2)]
    g5 = [gsum((lane // 4) % 2 == j) for j in range(2)]
    g_out = [gsum(lane % 4 == o) for o in range(4)]

    pix = jax.lax.broadcasted_iota(jnp.int32, (BM, D_IN), 1)
    quad = (pix // 28 >= 14).astype(jnp.int32) * 2 + (pix % 28 >= 14).astype(jnp.int32)
    gin_full = (jnp.where(quad == 0, g_in[0], 0.0) + jnp.where(quad == 1, g_in[1], 0.0)
                + jnp.where(quad == 2, g_in[2], 0.0) + jnp.where(quad == 3, g_in[3], 0.0))
    xg = x * gin_full

    def gate2(ga, gb, ncols):
        idx = jax.lax.broadcasted_iota(jnp.int32, (BM, ncols), 1)
        return jnp.where(idx < ncols // 2, ga, gb)

    def layer(h, W, b, gfull, act=True):
        y = jnp.dot(h.astype(bf16), W[...], preferred_element_type=f32) + b[...]
        if act:
            y = jnp.maximum(y, 0.0)
        return y * gfull

    h = layer(xg, W1, b1, gate2(g1[0], g1[1], H))
    h = layer(h, W2, b2, gate2(g2[0], g2[1], H))
    h = layer(h, W3, b3, gate2(g3[0], g3[1], H))
    h = layer(h, W4, b4, gate2(g4[0], g4[1], H))
    h = layer(h, W5, b5, gate2(g5[0], g5[1], H))

    oidx = jax.lax.broadcasted_iota(jnp.int32, (BM, D_OUT), 1)
    gout_full = (jnp.where(oidx < 256, g_out[0], 0.0)
                 + jnp.where((oidx >= 256) & (oidx < 512), g_out[1], 0.0)
                 + jnp.where((oidx >= 512) & (oidx < 768), g_out[2], 0.0)
                 + jnp.where(oidx >= 768, g_out[3], 0.0))
    out_ref[...] = layer(h, W6, b6, gout_full, act=False)


def kernel(x, W1, b1, W2, b2, W3, b3, W4, b4, W5, b5, W6, b6, Wr1, br1, Wr2, br2):
    wb = lambda w: w.astype(jnp.bfloat16)
    bb = lambda b: b.reshape(1, -1)

    def full(arr):
        return pl.BlockSpec(arr.shape, lambda i: (0, 0))

    ops = [wb(W1), bb(b1), wb(W2), bb(b2), wb(W3), bb(b3), wb(W4), bb(b4),
           wb(W5), bb(b5), wb(W6), bb(b6), wb(Wr1), bb(br1), wb(Wr2), bb(br2)]

    return pl.pallas_call(
        _mlp_body,
        grid=(B // BM,),
        in_specs=[pl.BlockSpec((BM, D_IN), lambda i: (i, 0))] + [full(a) for a in ops],
        out_specs=pl.BlockSpec((BM, D_OUT), lambda i: (i, 0)),
        out_shape=jax.ShapeDtypeStruct((B, D_OUT), jnp.float32),
    )(x, *ops)


# speedup vs baseline: 1.8628x; 1.0031x over previous
import jax
import jax.numpy as jnp
from jax.experimental import pallas as pl

B = 4096
D_IN = 784
H = 2048
D_OUT = 1024
RH = 256
NP = 512
BM = 512


def _mlp_body(x_ref, W1, b1, W2, b2, W3, b3, W4, b4, W5, b5, W6, b6,
              Wr1, br1, Wr2, br2, out_ref):
    f32 = jnp.float32
    bf16 = jnp.bfloat16
    x = x_ref[...]

    r = jnp.dot(x.astype(bf16), Wr1[...], preferred_element_type=f32) + br1[...]
    r = jnp.maximum(r, 0.0)
    logits = jnp.dot(r.astype(bf16), Wr2[...], preferred_element_type=f32) + br2[...]
    m = jnp.max(logits, axis=1, keepdims=True)
    e = jnp.exp(logits - m)
    inv_total = 1.0 / jnp.sum(e, axis=1, keepdims=True)

    lane = jax.lax.broadcasted_iota(jnp.int32, (BM, NP), 1)

    def gsum(mask):
        return jnp.sum(jnp.where(mask, e, 0.0), axis=1, keepdims=True) * inv_total

    g_in = [gsum(lane // 128 == i) for i in range(4)]
    g1 = [gsum((lane // 64) % 2 == j) for j in range(2)]
    g2 = [gsum((lane // 32) % 2 == j) for j in range(2)]
    g3 = [gsum((lane // 16) % 2 == j) for j in range(2)]
    g4 = [gsum((lane // 8) % 2 == j) for j in range(2)]
    g5 = [gsum((lane // 4) % 2 == j) for j in range(2)]
    g_out = [gsum(lane % 4 == o) for o in range(4)]

    pix = jax.lax.broadcasted_iota(jnp.int32, (BM, D_IN), 1)
    quad = (pix // 28 >= 14).astype(jnp.int32) * 2 + (pix % 28 >= 14).astype(jnp.int32)
    gin_full = (jnp.where(quad == 0, g_in[0], 0.0) + jnp.where(quad == 1, g_in[1], 0.0)
                + jnp.where(quad == 2, g_in[2], 0.0) + jnp.where(quad == 3, g_in[3], 0.0))
    xg = (x * gin_full).astype(bf16)

    def layer(h, W, b, ga, gb):
        y = jnp.dot(h, W[...], preferred_element_type=f32)
        n = y.shape[1] // 2
        ya = (jnp.maximum(y[:, :n] + b[:, :n], 0.0) * ga).astype(bf16)
        yb = (jnp.maximum(y[:, n:] + b[:, n:], 0.0) * gb).astype(bf16)
        return jnp.concatenate([ya, yb], axis=1)

    h = layer(xg, W1, b1, g1[0], g1[1])
    h = layer(h, W2, b2, g2[0], g2[1])
    h = layer(h, W3, b3, g3[0], g3[1])
    h = layer(h, W4, b4, g4[0], g4[1])
    h = layer(h, W5, b5, g5[0], g5[1])

    y = jnp.dot(h, W6[...], preferred_element_type=f32) + b6[...]
    q = D_OUT // 4
    out_ref[...] = jnp.concatenate(
        [y[:, o * q:(o + 1) * q] * g_out[o] for o in range(4)], axis=1)


def kernel(x, W1, b1, W2, b2, W3, b3, W4, b4, W5, b5, W6, b6, Wr1, br1, Wr2, br2):
    wb = lambda w: w.astype(jnp.bfloat16)
    bb = lambda b: b.reshape(1, -1)

    def full(arr):
        return pl.BlockSpec(arr.shape, lambda i: (0, 0))

    ops = [wb(W1), bb(b1), wb(W2), bb(b2), wb(W3), bb(b3), wb(W4), bb(b4),
           wb(W5), bb(b5), wb(W6), bb(b6), wb(Wr1), bb(br1), wb(Wr2), bb(br2)]

    return pl.pallas_call(
        _mlp_body,
        grid=(B // BM,),
        in_specs=[pl.BlockSpec((BM, D_IN), lambda i: (i, 0))] + [full(a) for a in ops],
        out_specs=pl.BlockSpec((BM, D_OUT), lambda i: (i, 0)),
        out_shape=jax.ShapeDtypeStruct((B, D_OUT), jnp.float32),
    )(x, *ops)
